# baseline (speedup 1.0000x reference)
import functools

import jax
import jax.numpy as jnp
from jax import lax
from jax.experimental import pallas as pl
from jax.experimental.pallas import tpu as pltpu
from jax.experimental.pallas import tpu_sc as plsc

NC = 2
NS = 16
NW = NC * NS
LANES = 16
CH = 128
SLOPE = (1.0 / 8.0 + 1.0 / 3.0) / 2.0
F = 128


def _sc_mesh():
    return plsc.VectorSubcoreMesh(core_axis_name="c", subcore_axis_name="s")


def _make_deg_kernel(npad, k):
    npt = npad // NS

    @functools.partial(
        pl.kernel,
        out_type=jax.ShapeDtypeStruct((NC, npad), jnp.float32),
        mesh=_sc_mesh(),
        scratch_types=[
            pltpu.VMEM((k, CH), jnp.int32),
            pltpu.VMEM((CH,), jnp.float32),
            pltpu.VMEM((npt,), jnp.float32),
            pltpu.VMEM_SHARED((npad,), jnp.float32),
        ],
    )
    def deg_k(dst_hbm, out_hbm, dst_v, ones_v, buf_v, deg_s):
        c = lax.axis_index("c")
        s = lax.axis_index("s")
        w = c * NS + s
        base = s * npt
        one16 = jnp.full((LANES,), 1.0, jnp.float32)
        zero16 = jnp.zeros((LANES,), jnp.float32)

        @pl.loop(0, CH // LANES)
        def _(i):
            ones_v[pl.ds(i * LANES, LANES)] = one16

        @pl.loop(0, npt // LANES)
        def _(i):
            buf_v[pl.ds(i * LANES, LANES)] = zero16

        pltpu.sync_copy(buf_v, deg_s.at[pl.ds(base, npt)])
        pltpu.sync_copy(dst_hbm.at[w], dst_v)
        plsc.subcore_barrier()

        @pl.loop(0, k)
        def _(j):
            pltpu.sync_copy(ones_v, deg_s.at[dst_v.at[j]], add=True)

        plsc.subcore_barrier()
        pltpu.sync_copy(deg_s.at[pl.ds(base, npt)], buf_v)
        pltpu.sync_copy(buf_v, out_hbm.at[c, pl.ds(base, npt)])

    return deg_k


def _make_agg_kernel(npad, k):
    npt = npad // NS
    cpt = npt // CH

    @functools.partial(
        pl.kernel,
        out_type=jax.ShapeDtypeStruct((NC, npad, F), jnp.float32),
        mesh=_sc_mesh(),
        scratch_types=[
            pltpu.VMEM((k // 2, CH), jnp.int32),
            pltpu.VMEM((k // 2, CH), jnp.int32),
            pltpu.VMEM((2, CH, F), jnp.float32),
            pltpu.VMEM_SHARED((npad, F), jnp.float32),
            pltpu.SemaphoreType.DMA,
            pltpu.SemaphoreType.DMA,
        ],
    )
    def agg_k(src_hbm, dst_hbm, hs_hbm, out_hbm,
              src_v, dst_v, rows2_v, acc_s, sem0, sem1):
        rows_v = rows2_v.at[0]
        c = lax.axis_index("c")
        s = lax.axis_index("s")
        w = c * NS + s
        base = s * npt
        zero16 = jnp.zeros((LANES,), jnp.float32)

        @pl.loop(0, CH)
        def _(r):
            @pl.loop(0, F // LANES)
            def _(l2):
                rows_v[r, pl.ds(l2 * LANES, LANES)] = zero16

        @pl.loop(0, cpt)
        def _(i):
            pltpu.sync_copy(rows_v, acc_s.at[pl.ds(base + i * CH, CH)])

        plsc.subcore_barrier()

        kh = k // 2
        for h in range(2):
            pltpu.sync_copy(src_hbm.at[w, pl.ds(h * kh, kh)], src_v)
            pltpu.sync_copy(dst_hbm.at[w, pl.ds(h * kh, kh)], dst_v)

            @pl.loop(0, kh // 2)
            def _(j):
                g0 = pltpu.async_copy(
                    hs_hbm.at[src_v.at[2 * j]], rows2_v.at[0], sem0)
                g1 = pltpu.async_copy(
                    hs_hbm.at[src_v.at[2 * j + 1]], rows2_v.at[1], sem1)
                g0.wait()
                pltpu.sync_copy(
                    rows2_v.at[0], acc_s.at[dst_v.at[2 * j]], add=True)
                g1.wait()
                pltpu.sync_copy(
                    rows2_v.at[1], acc_s.at[dst_v.at[2 * j + 1]], add=True)

        plsc.subcore_barrier()

        @pl.loop(0, cpt)
        def _(i):
            pltpu.sync_copy(acc_s.at[pl.ds(base + i * CH, CH)], rows_v)
            pltpu.sync_copy(rows_v, out_hbm.at[c, pl.ds(base + i * CH, CH)])

    return agg_k


def _scale_fn(degT_ref, x_ref, w_ref, hs_ref, dinv_ref):
    dg = degT_ref[...]
    d = dg[:, 0:1] + dg[:, 1:2] + 1.0
    dinv = lax.rsqrt(d)
    h = jnp.dot(x_ref[...], w_ref[...], preferred_element_type=jnp.float32)
    hs_ref[...] = h * dinv
    dinv_ref[...] = dinv


def _layer_fn(p_ref, hs_ref, dinv_ref, b_ref, w_ref, out_ref):
    dinv = dinv_ref[...]
    v = (p_ref[0] + p_ref[1] + hs_ref[...]) * dinv + b_ref[...]
    g = jnp.where(v >= 0, v, v * SLOPE)
    out_ref[...] = jnp.dot(g, w_ref[...], preferred_element_type=jnp.float32) * dinv


def _final_fn(q_ref, hs_ref, dinv_ref, b_ref, w_ref, bl_ref, out_ref):
    v = (q_ref[0] + q_ref[1] + hs_ref[...]) * dinv_ref[...] + b_ref[...]
    g = jnp.where(v >= 0, v, v * SLOPE)
    out_ref[...] = jnp.dot(g, w_ref[...], preferred_element_type=jnp.float32) + bl_ref[...]


def kernel(x, edge_index, W1, b1, W2, b2, Wl, bl):
    n, d_in = x.shape
    e = edge_index.shape[1]
    npad = ((n + 1 + 1023) // 1024) * 1024
    bm = 1024
    grid = (npad // bm,)

    k = -(-e // (NW * CH))
    k = (k + 3) // 4 * 4
    epad = NW * k * CH

    src = edge_index[0]
    dst = edge_index[1]

    def _shard(a, fill):
        ap = jnp.concatenate([a, jnp.full((epad - e,), fill, a.dtype)])
        return ap.reshape(NW, k, CH)

    srcp = _shard(src, 0)
    dstp = _shard(dst, n)
    xp = jnp.pad(x, ((0, npad - n), (0, 0)))

    degp = _make_deg_kernel(npad, k)(dstp)
    degT = jnp.stack([degp[0], degp[1]], axis=1)

    h1s, dinv = pl.pallas_call(
        _scale_fn,
        grid=grid,
        in_specs=[
            pl.BlockSpec((bm, 2), lambda i: (i, 0)),
            pl.BlockSpec((bm, d_in), lambda i: (i, 0)),
            pl.BlockSpec((d_in, F), lambda i: (0, 0)),
        ],
        out_specs=[
            pl.BlockSpec((bm, F), lambda i: (i, 0)),
            pl.BlockSpec((bm, 1), lambda i: (i, 0)),
        ],
        out_shape=[
            jax.ShapeDtypeStruct((npad, F), jnp.float32),
            jax.ShapeDtypeStruct((npad, 1), jnp.float32),
        ],
    )(degT, xp, W1)

    agg = _make_agg_kernel(npad, k)

    p = agg(srcp, dstp, h1s)
    h2s = pl.pallas_call(
        _layer_fn,
        grid=grid,
        in_specs=[
            pl.BlockSpec((NC, bm, F), lambda i: (0, i, 0)),
            pl.BlockSpec((bm, F), lambda i: (i, 0)),
            pl.BlockSpec((bm, 1), lambda i: (i, 0)),
            pl.BlockSpec((1, F), lambda i: (0, 0)),
            pl.BlockSpec((F, F), lambda i: (0, 0)),
        ],
        out_specs=pl.BlockSpec((bm, F), lambda i: (i, 0)),
        out_shape=jax.ShapeDtypeStruct((npad, F), jnp.float32),
    )(p, h1s, dinv, b1.reshape(1, F), W2)

    q = agg(srcp, dstp, h2s)
    outp = pl.pallas_call(
        _final_fn,
        grid=grid,
        in_specs=[
            pl.BlockSpec((NC, bm, F), lambda i: (0, i, 0)),
            pl.BlockSpec((bm, F), lambda i: (i, 0)),
            pl.BlockSpec((bm, 1), lambda i: (i, 0)),
            pl.BlockSpec((1, F), lambda i: (0, 0)),
            pl.BlockSpec((F, F), lambda i: (0, 0)),
            pl.BlockSpec((1, F), lambda i: (0, 0)),
        ],
        out_specs=pl.BlockSpec((bm, F), lambda i: (i, 0)),
        out_shape=jax.ShapeDtypeStruct((npad, F), jnp.float32),
    )(q, h2s, dinv, b2.reshape(1, F), Wl, bl.reshape(1, F))

    return outp[:n]


# speedup vs baseline: 1.5221x; 1.5221x over previous
import functools

import jax
import jax.numpy as jnp
from jax import lax
from jax.experimental import pallas as pl
from jax.experimental.pallas import tpu as pltpu
from jax.experimental.pallas import tpu_sc as plsc

NC = 2
NS = 16
NW = NC * NS
LANES = 16
CH = 128
SLOPE = (1.0 / 8.0 + 1.0 / 3.0) / 2.0
F = 128


def _sc_mesh():
    return plsc.VectorSubcoreMesh(core_axis_name="c", subcore_axis_name="s")


def _make_deg_kernel(npad, k):
    npt = npad // NS

    @functools.partial(
        pl.kernel,
        out_type=jax.ShapeDtypeStruct((NC, npad), jnp.float32),
        mesh=_sc_mesh(),
        scratch_types=[
            pltpu.VMEM((k, CH), jnp.int32),
            pltpu.VMEM((CH,), jnp.float32),
            pltpu.VMEM((npt,), jnp.float32),
            pltpu.VMEM_SHARED((npad,), jnp.float32),
        ],
    )
    def deg_k(dst_hbm, out_hbm, dst_v, ones_v, buf_v, deg_s):
        c = lax.axis_index("c")
        s = lax.axis_index("s")
        w = c * NS + s
        base = s * npt
        one16 = jnp.full((LANES,), 1.0, jnp.float32)
        zero16 = jnp.zeros((LANES,), jnp.float32)

        @pl.loop(0, CH // LANES)
        def _(i):
            ones_v[pl.ds(i * LANES, LANES)] = one16

        @pl.loop(0, npt // LANES)
        def _(i):
            buf_v[pl.ds(i * LANES, LANES)] = zero16

        pltpu.sync_copy(buf_v, deg_s.at[pl.ds(base, npt)])
        pltpu.sync_copy(dst_hbm.at[w], dst_v)
        plsc.subcore_barrier()

        @pl.loop(0, k)
        def _(j):
            pltpu.sync_copy(ones_v, deg_s.at[dst_v.at[j]], add=True)

        plsc.subcore_barrier()
        pltpu.sync_copy(deg_s.at[pl.ds(base, npt)], buf_v)
        pltpu.sync_copy(buf_v, out_hbm.at[c, pl.ds(base, npt)])

    return deg_k


def _make_agg_kernel(npad, k):
    npt = npad // NS
    cpt = npt // CH

    @functools.partial(
        pl.kernel,
        out_type=jax.ShapeDtypeStruct((NC, npad, F), jnp.float32),
        mesh=_sc_mesh(),
        scratch_types=[
            pltpu.VMEM((k, CH), jnp.int32),
            pltpu.VMEM((k, CH), jnp.int32),
            pltpu.VMEM((CH, F), jnp.float32),
            pltpu.VMEM_SHARED((npad, F), jnp.float32),
        ],
    )
    def agg_k(src_hbm, dst_hbm, hs_hbm, out_hbm,
              src_v, dst_v, rows_v, acc_s):
        c = lax.axis_index("c")
        s = lax.axis_index("s")
        w = c * NS + s
        base = s * npt
        zero16 = jnp.zeros((LANES,), jnp.float32)

        @pl.loop(0, CH)
        def _(r):
            @pl.loop(0, F // LANES)
            def _(l2):
                rows_v[r, pl.ds(l2 * LANES, LANES)] = zero16

        @pl.loop(0, cpt)
        def _(i):
            pltpu.sync_copy(rows_v, acc_s.at[pl.ds(base + i * CH, CH)])

        pltpu.sync_copy(src_hbm.at[w], src_v)
        pltpu.sync_copy(dst_hbm.at[w], dst_v)
        plsc.subcore_barrier()

        @pl.loop(0, k)
        def _(j):
            pltpu.sync_copy(hs_hbm.at[src_v.at[j]], rows_v)
            pltpu.sync_copy(rows_v, acc_s.at[dst_v.at[j]], add=True)

        plsc.subcore_barrier()

        pltpu.sync_copy(acc_s.at[pl.ds(base, npt)],
                        out_hbm.at[c, pl.ds(base, npt)])

    return agg_k


def _scale_fn(degT_ref, x_ref, w_ref, hs_ref, dinv_ref):
    dg = degT_ref[...]
    d = dg[:, 0:1] + dg[:, 1:2] + 1.0
    dinv = lax.rsqrt(d)
    h = jnp.dot(x_ref[...], w_ref[...], preferred_element_type=jnp.float32)
    hs_ref[...] = h * dinv
    dinv_ref[...] = dinv


def _layer_fn(p_ref, hs_ref, dinv_ref, b_ref, w_ref, out_ref):
    dinv = dinv_ref[...]
    v = (p_ref[0] + p_ref[1] + hs_ref[...]) * dinv + b_ref[...]
    g = jnp.where(v >= 0, v, v * SLOPE)
    out_ref[...] = jnp.dot(g, w_ref[...], preferred_element_type=jnp.float32) * dinv


def _final_fn(q_ref, hs_ref, dinv_ref, b_ref, w_ref, bl_ref, out_ref):
    v = (q_ref[0] + q_ref[1] + hs_ref[...]) * dinv_ref[...] + b_ref[...]
    g = jnp.where(v >= 0, v, v * SLOPE)
    out_ref[...] = jnp.dot(g, w_ref[...], preferred_element_type=jnp.float32) + bl_ref[...]


def kernel(x, edge_index, W1, b1, W2, b2, Wl, bl):
    n, d_in = x.shape
    e = edge_index.shape[1]
    npad = ((n + 1 + 1023) // 1024) * 1024
    bm = 1024
    grid = (npad // bm,)

    k = -(-e // (NW * CH))
    epad = NW * k * CH

    src = edge_index[0]
    dst = edge_index[1]

    def _shard(a, fill):
        ap = jnp.concatenate([a, jnp.full((epad - e,), fill, a.dtype)])
        return ap.reshape(NW, k, CH)

    srcp = _shard(src, 0)
    dstp = _shard(dst, n)
    xp = jnp.pad(x, ((0, npad - n), (0, 0)))

    degp = _make_deg_kernel(npad, k)(dstp)
    degT = jnp.stack([degp[0], degp[1]], axis=1)

    h1s, dinv = pl.pallas_call(
        _scale_fn,
        grid=grid,
        in_specs=[
            pl.BlockSpec((bm, 2), lambda i: (i, 0)),
            pl.BlockSpec((bm, d_in), lambda i: (i, 0)),
            pl.BlockSpec((d_in, F), lambda i: (0, 0)),
        ],
        out_specs=[
            pl.BlockSpec((bm, F), lambda i: (i, 0)),
            pl.BlockSpec((bm, 1), lambda i: (i, 0)),
        ],
        out_shape=[
            jax.ShapeDtypeStruct((npad, F), jnp.float32),
            jax.ShapeDtypeStruct((npad, 1), jnp.float32),
        ],
    )(degT, xp, W1)

    agg = _make_agg_kernel(npad, k)

    p = agg(srcp, dstp, h1s)
    h2s = pl.pallas_call(
        _layer_fn,
        grid=grid,
        in_specs=[
            pl.BlockSpec((NC, bm, F), lambda i: (0, i, 0)),
            pl.BlockSpec((bm, F), lambda i: (i, 0)),
            pl.BlockSpec((bm, 1), lambda i: (i, 0)),
            pl.BlockSpec((1, F), lambda i: (0, 0)),
            pl.BlockSpec((F, F), lambda i: (0, 0)),
        ],
        out_specs=pl.BlockSpec((bm, F), lambda i: (i, 0)),
        out_shape=jax.ShapeDtypeStruct((npad, F), jnp.float32),
    )(p, h1s, dinv, b1.reshape(1, F), W2)

    q = agg(srcp, dstp, h2s)
    outp = pl.pallas_call(
        _final_fn,
        grid=grid,
        in_specs=[
            pl.BlockSpec((NC, bm, F), lambda i: (0, i, 0)),
            pl.BlockSpec((bm, F), lambda i: (i, 0)),
            pl.BlockSpec((bm, 1), lambda i: (i, 0)),
            pl.BlockSpec((1, F), lambda i: (0, 0)),
            pl.BlockSpec((F, F), lambda i: (0, 0)),
            pl.BlockSpec((1, F), lambda i: (0, 0)),
        ],
        out_specs=pl.BlockSpec((bm, F), lambda i: (i, 0)),
        out_shape=jax.ShapeDtypeStruct((npad, F), jnp.float32),
    )(q, h2s, dinv, b2.reshape(1, F), Wl, bl.reshape(1, F))

    return outp[:n]
